# Initial kernel scaffold; baseline (speedup 1.0000x reference)
import jax
import jax.numpy as jnp
from jax.experimental import pallas as pl


def kernel(x, edge_index, W1, b1, W2, b2):
    raise NotImplementedError("write your pallas kernel here")


# speedup vs baseline: 15.0401x; 15.0401x over previous
import functools

import jax
import jax.numpy as jnp
from jax import lax
from jax.experimental import pallas as pl
from jax.experimental.pallas import tpu as pltpu
from jax.experimental.pallas import tpu_sc as plsc

N = 10000
E = 320000
DIN = 128
DH = 16
DOUT = 128

NC = 2
NS = 16
L = 16

NPAD = 10240
RPT = NPAD // NS
EPAD = 327680
CH = 128
EPW = EPAD // (NC * NS)
NCH = EPW // CH
NCHA = (EPAD // NS) // CH

_mesh = plsc.VectorSubcoreMesh(core_axis_name="c", subcore_axis_name="s",
                               num_cores=NC, num_subcores=NS)


@functools.partial(
    pl.kernel,
    out_type=jax.ShapeDtypeStruct((NC, NPAD, L), jnp.float32),
    mesh=_mesh,
    compiler_params=pltpu.CompilerParams(use_tc_tiling_on_sc=False),
    scratch_types=[
        pltpu.VMEM_SHARED((NPAD, L), jnp.float32),
        pltpu.VMEM((NCHA, CH), jnp.int32),
        pltpu.VMEM((CH, L), jnp.float32),
        pltpu.VMEM((RPT, L), jnp.float32),
        pltpu.SemaphoreType.DMA,
    ],
)
def _deg_kernel(edgesA, degs_out, hist_sh, idx_v, ones_v, buf_v, sem):
    c = lax.axis_index("c")
    s = lax.axis_index("s")
    rows = pl.ds(s * RPT, RPT)

    @pl.loop(0, CH)
    def _(r):
        ones_v[r, :] = jnp.full((L,), 1.0, jnp.float32)

    @pl.loop(0, RPT)
    def _(r):
        buf_v[r, :] = jnp.zeros((L,), jnp.float32)

    pltpu.sync_copy(buf_v, hist_sh.at[rows])
    plsc.subcore_barrier()

    pltpu.sync_copy(edgesA.at[c].at[s], idx_v)

    @pl.loop(0, NCHA)
    def _(j):
        pltpu.sync_copy(ones_v, hist_sh.at[idx_v.at[j]], add=True)

    plsc.subcore_barrier()

    pltpu.sync_copy(hist_sh.at[rows], degs_out.at[c].at[rows])


def _agg_phase(table_sh, agg_sh, src_hbm, dst_hbm, srcv, dstv, rows_v, sem,
               w, buf_v, part_out, c, s, rows):
    @pl.loop(0, RPT)
    def _(r):
        buf_v[r, :] = jnp.zeros((L,), jnp.float32)

    pltpu.sync_copy(buf_v, agg_sh.at[rows])
    plsc.subcore_barrier()

    pltpu.sync_copy(src_hbm.at[w], srcv)
    pltpu.sync_copy(dst_hbm.at[w], dstv)

    @pl.loop(0, NCH)
    def _(j):
        pltpu.async_copy(table_sh.at[srcv.at[j]], rows_v, sem).wait()
        pltpu.sync_copy(rows_v, agg_sh.at[dstv.at[j]], add=True)

    plsc.subcore_barrier()
    pltpu.sync_copy(agg_sh.at[rows], buf_v)
    pltpu.sync_copy(buf_v, part_out.at[c].at[rows])


_agg_scratch = [
    pltpu.VMEM_SHARED((NPAD, L), jnp.float32),
    pltpu.VMEM_SHARED((NPAD, L), jnp.float32),
    pltpu.VMEM((RPT, L), jnp.float32),
    pltpu.VMEM((RPT, L), jnp.float32),
    pltpu.VMEM((NCH, CH), jnp.int32),
    pltpu.VMEM((NCH, CH), jnp.int32),
    pltpu.VMEM((CH, L), jnp.float32),
    pltpu.SemaphoreType.DMA,
]


@functools.partial(
    pl.kernel,
    out_type=jax.ShapeDtypeStruct((NC, NPAD, L), jnp.float32),
    mesh=_mesh,
    compiler_params=pltpu.CompilerParams(use_tc_tiling_on_sc=False),
    scratch_types=_agg_scratch,
)
def _layer1_kernel(xwn, src_hbm, dst_hbm, part_out, table_sh, agg_sh,
                   a_v, b_v, srcv, dstv, rows_v, sem):
    c = lax.axis_index("c")
    s = lax.axis_index("s")
    w = c * NS + s
    rows = pl.ds(s * RPT, RPT)

    pltpu.sync_copy(xwn.at[rows], table_sh.at[rows])

    _agg_phase(table_sh, agg_sh, src_hbm, dst_hbm, srcv, dstv, rows_v, sem,
               w, b_v, part_out, c, s, rows)


@functools.partial(
    pl.kernel,
    out_type=jax.ShapeDtypeStruct((NC, NPAD, L), jnp.float32),
    mesh=_mesh,
    compiler_params=pltpu.CompilerParams(use_tc_tiling_on_sc=False),
    scratch_types=_agg_scratch + [pltpu.VMEM((L,), jnp.float32)],
)
def _layer2_kernel(p0, p1, nsrc, ndst, b1, src_hbm, dst_hbm, part_out,
                   table_sh, agg_sh, a_v, b_v, srcv, dstv, rows_v, sem, b1_v):
    c = lax.axis_index("c")
    s = lax.axis_index("s")
    w = c * NS + s
    rows = pl.ds(s * RPT, RPT)

    pltpu.sync_copy(b1, b1_v)
    bias = b1_v[...]

    pltpu.sync_copy(p0.at[rows], a_v)
    pltpu.sync_copy(p1.at[rows], b_v)

    @pl.loop(0, RPT)
    def _(r):
        a_v[r, :] = a_v[r, :] + b_v[r, :]

    pltpu.sync_copy(ndst.at[rows], b_v)

    @pl.loop(0, RPT)
    def _(r):
        a_v[r, :] = jnp.maximum(a_v[r, :] * b_v[r, :] + bias, 0.0)

    pltpu.sync_copy(nsrc.at[rows], b_v)

    @pl.loop(0, RPT)
    def _(r):
        a_v[r, :] = a_v[r, :] * b_v[r, :]

    pltpu.sync_copy(a_v, table_sh.at[rows])

    _agg_phase(table_sh, agg_sh, src_hbm, dst_hbm, srcv, dstv, rows_v, sem,
               w, b_v, part_out, c, s, rows)


_RB = 1024


def _mm1_body(x_ref, w_ref, ds_ref, dd_ref, xwn_ref, ns_ref, nd_ref):
    ns = jnp.where(ds_ref[...] > 0.0, lax.rsqrt(ds_ref[...]), 1.0)
    nd = jnp.where(dd_ref[...] > 0.0, lax.rsqrt(dd_ref[...]), 1.0)
    xw = jnp.dot(x_ref[...], w_ref[...], preferred_element_type=jnp.float32)
    xwn_ref[...] = xw * ns
    ns_ref[...] = ns
    nd_ref[...] = nd


def _mm1(x_pad, W1, dsrc, ddst):
    return pl.pallas_call(
        _mm1_body,
        grid=(NPAD // _RB,),
        in_specs=[
            pl.BlockSpec((_RB, DIN), lambda i: (i, 0)),
            pl.BlockSpec((DIN, DH), lambda i: (0, 0)),
            pl.BlockSpec((_RB, DH), lambda i: (i, 0)),
            pl.BlockSpec((_RB, DH), lambda i: (i, 0)),
        ],
        out_specs=[
            pl.BlockSpec((_RB, DH), lambda i: (i, 0)),
            pl.BlockSpec((_RB, DH), lambda i: (i, 0)),
            pl.BlockSpec((_RB, DH), lambda i: (i, 0)),
        ],
        out_shape=[
            jax.ShapeDtypeStruct((NPAD, DH), jnp.float32),
            jax.ShapeDtypeStruct((NPAD, DH), jnp.float32),
            jax.ShapeDtypeStruct((NPAD, DH), jnp.float32),
        ],
    )(x_pad, W1, dsrc, ddst)


def _mm2_body(a_ref, b_ref, n_ref, w_ref, bias_ref, o_ref):
    h = (a_ref[...] + b_ref[...]) * n_ref[...]
    o_ref[...] = jnp.dot(h, w_ref[...],
                         preferred_element_type=jnp.float32) + bias_ref[...]


def _mm2(p0, p1, ndst, W2, b2):
    return pl.pallas_call(
        _mm2_body,
        grid=(NPAD // _RB,),
        in_specs=[
            pl.BlockSpec((_RB, DH), lambda i: (i, 0)),
            pl.BlockSpec((_RB, DH), lambda i: (i, 0)),
            pl.BlockSpec((_RB, DH), lambda i: (i, 0)),
            pl.BlockSpec((DH, DOUT), lambda i: (0, 0)),
            pl.BlockSpec((1, DOUT), lambda i: (0, 0)),
        ],
        out_specs=pl.BlockSpec((_RB, DOUT), lambda i: (i, 0)),
        out_shape=jax.ShapeDtypeStruct((NPAD, DOUT), jnp.float32),
    )(p0, p1, ndst, W2, b2.reshape(1, DOUT))


@jax.jit
def kernel(x, edge_index, W1, b1, W2, b2):
    src = edge_index[0]
    dst = edge_index[1]
    pad = jnp.full((EPAD - E,), N, jnp.int32)
    src_p = jnp.concatenate([src, pad])
    dst_p = jnp.concatenate([dst, pad])
    srcC = src_p.reshape(NC * NS, NCH, CH)
    dstC = dst_p.reshape(NC * NS, NCH, CH)
    edgesA = jnp.stack([src_p, dst_p]).reshape(NC, NS, NCHA, CH)
    x_pad = jnp.pad(x, ((0, NPAD - N), (0, 0)))

    degs = _deg_kernel(edgesA)
    dsrc, ddst = degs[0], degs[1]
    xwn, nsrc, ndst = _mm1(x_pad, W1, dsrc, ddst)
    p1 = _layer1_kernel(xwn, srcC, dstC)
    p2 = _layer2_kernel(p1[0], p1[1], nsrc, ndst, b1, srcC, dstC)
    out = _mm2(p2[0], p2[1], ndst, W2, b2)
    return out[:N]
